# baseline (speedup 1.0000x reference)
import jax
import jax.numpy as jnp
from jax.experimental import pallas as pl
from jax.experimental.pallas import tpu as pltpu

N_UP = 32
N_DOWN = 32
NE = N_UP + N_DOWN
NA = 16
DF = 4
KD = 8
ED = 16
HI = 6
NI = 2
BBH = 8
RB = 512

_LOG2 = 0.6931471805599453

_O_W1 = 0
_O_B1 = _O_W1 + NI * 3 * DF * HI
_O_W2 = _O_B1 + NI * 3 * HI
_O_B2 = _O_W2 + NI * 3 * HI * KD
_O_HW = _O_B2 + NI * 3 * KD
_O_HB = _O_HW + NI * ED * KD
_O_GW = _O_HB + NI * KD
_O_GB = _O_GW + NI * KD * ED
_O_X0 = _O_GB + NI * ED
_O_OW = _O_X0 + ED


def _ssp(v):
    return jax.nn.softplus(v) - _LOG2


def _wnet(d, n, s, th_ref):
    th = lambda i: th_ref[i]
    hid = []
    for h in range(HI):
        a = d[0] * th(_O_W1 + ((n * 3 + s) * DF + 0) * HI + h)
        for f in range(1, DF):
            a = a + d[f] * th(_O_W1 + ((n * 3 + s) * DF + f) * HI + h)
        hid.append(_ssp(a + th(_O_B1 + (n * 3 + s) * HI + h)))
    ws = []
    for k in range(KD):
        a = hid[0] * th(_O_W2 + ((n * 3 + s) * HI + 0) * KD + k)
        for h in range(1, HI):
            a = a + hid[h] * th(_O_W2 + ((n * 3 + s) * HI + h) * KD + k)
        ws.append(_ssp(a + th(_O_B2 + (n * 3 + s) * KD + k)))
    return ws


def _body_nuc(dn_ref, th_ref, y2_ref, out_ref):
    d = [dn_ref[f] for f in range(DF)]
    ll = jax.lax.broadcasted_iota(jnp.int32, (128, KD), 0)
    cc = jax.lax.broadcasted_iota(jnp.int32, (128, KD), 1)
    s16 = (ll // NA == cc).astype(jnp.float32)
    for n in range(NI):
        w = _wnet(d, n, 2, th_ref)
        for k in range(KD):
            partial = w[k] * y2_ref[k][None, :]
            out_ref[n, k] = jnp.dot(partial, s16,
                                    preferred_element_type=jnp.float32)


def _body_main(de_ref, zn_ref, th_ref, out_ref):
    th = lambda i: th_ref[i]
    de = [de_ref[f] for f in range(DF)]

    ii = jax.lax.broadcasted_iota(jnp.int32, (NE, 2 * NE), 0)
    ll = jax.lax.broadcasted_iota(jnp.int32, (NE, 2 * NE), 1)
    jm = ll % NE
    sameb = (ii < N_UP) == (jm < N_UP)
    diagm = ii == jm
    l1 = jax.lax.broadcasted_iota(jnp.int32, (2 * NE, 2), 0)
    c1 = jax.lax.broadcasted_iota(jnp.int32, (2 * NE, 2), 1)
    s2 = (l1 // NE == c1).astype(jnp.float32)

    x = [jnp.full((BBH, 2 * NE), th(_O_X0 + e), jnp.float32)
         for e in range(ED)]

    for n in range(NI):
        w_same = _wnet(de, n, 0, th_ref)
        w_anti = _wnet(de, n, 1, th_ref)

        hk = []
        for k in range(KD):
            a = x[0] * th(_O_HW + (n * ED + 0) * KD + k)
            for e in range(1, ED):
                a = a + x[e] * th(_O_HW + (n * ED + e) * KD + k)
            hk.append(a + th(_O_HB + n * KD + k))

        zk = []
        for k in range(KD):
            wc = jnp.where(diagm[None], 0.0,
                           jnp.where(sameb[None], w_same[k], w_anti[k]))
            partial = wc * hk[k][:, None, :]
            z0 = jnp.sum(partial[:, :, :NE], axis=-1)
            z1 = jnp.sum(partial[:, :, NE:], axis=-1)
            zk.append(jnp.concatenate([z0, z1], axis=-1) + zn_ref[n, k])

        xn = []
        for e in range(ED):
            a = zk[0] * th(_O_GW + (n * KD + 0) * ED + e)
            for k in range(1, KD):
                a = a + zk[k] * th(_O_GW + (n * KD + k) * ED + e)
            xn.append(x[e] + a + th(_O_GB + n * ED + e))
        x = xn

    tot = jnp.dot(x[0], s2, preferred_element_type=jnp.float32) * th(_O_OW + 0)
    for e in range(1, ED):
        tot = tot + jnp.dot(x[e], s2,
                            preferred_element_type=jnp.float32) * th(_O_OW + e)
    out_ref[...] = tot[None]


def kernel(dists_nuc, dists_elec, W1, b1, W2, b2, hW, hb, gW, gb, X0, Y, oW):
    Bd = dists_elec.shape[0]
    theta = jnp.concatenate([
        W1.ravel(), b1.ravel(), W2.ravel(), b2.ravel(),
        hW.ravel(), hb.ravel(), gW.ravel(), gb.ravel(),
        X0.ravel(), oW.ravel(),
    ]).astype(jnp.float32)

    X = Bd * NE * NA // 128
    rb = min(RB, X)
    dn_f = jnp.transpose(dists_nuc, (3, 0, 1, 2)).reshape(DF, X, 128)
    y2 = jnp.tile(Y.T, (1, 128 // NA))
    znuc = pl.pallas_call(
        _body_nuc,
        grid=(X // rb,),
        in_specs=[
            pl.BlockSpec((DF, rb, 128), lambda i: (0, i, 0)),
            pl.BlockSpec(memory_space=pltpu.SMEM),
            pl.BlockSpec((KD, 128), lambda i: (0, 0)),
        ],
        out_specs=pl.BlockSpec((NI, KD, rb, KD), lambda i: (0, 0, i, 0)),
        out_shape=jax.ShapeDtypeStruct((NI, KD, X, KD), jnp.float32),
    )(dn_f, theta, y2)
    zn_m = znuc.reshape(NI, KD, Bd // 2, 2 * NE)

    de_m = jnp.transpose(
        dists_elec.reshape(Bd // 2, 2, NE, NE, DF), (4, 0, 2, 1, 3)
    ).reshape(DF, Bd // 2, NE, 2 * NE)

    G = Bd // 2 // BBH
    out = pl.pallas_call(
        _body_main,
        grid=(G,),
        in_specs=[
            pl.BlockSpec((DF, BBH, NE, 2 * NE), lambda i: (0, i, 0, 0)),
            pl.BlockSpec((NI, KD, BBH, 2 * NE), lambda i: (0, 0, i, 0)),
            pl.BlockSpec(memory_space=pltpu.SMEM),
        ],
        out_specs=pl.BlockSpec((1, BBH, 2), lambda i: (i, 0, 0)),
        out_shape=jax.ShapeDtypeStruct((G, BBH, 2), jnp.float32),
    )(de_m, zn_m, theta)
    return out.reshape(Bd)


# speedup vs baseline: 1.3809x; 1.3809x over previous
import jax
import jax.numpy as jnp
from jax.experimental import pallas as pl
from jax.experimental.pallas import tpu as pltpu

N_UP = 32
N_DOWN = 32
NE = N_UP + N_DOWN
NA = 16
DF = 4
KD = 8
ED = 16
HI = 6
NI = 2
BBH = 8
RB = 512

_LOG2 = 0.6931471805599453

_O_W1 = 0
_O_B1 = _O_W1 + NI * 3 * DF * HI
_O_W2 = _O_B1 + NI * 3 * HI
_O_B2 = _O_W2 + NI * 3 * HI * KD
_O_HW = _O_B2 + NI * 3 * KD
_O_HB = _O_HW + NI * ED * KD
_O_GW = _O_HB + NI * KD
_O_GB = _O_GW + NI * KD * ED
_O_X0 = _O_GB + NI * ED
_O_OW = _O_X0 + ED


def _ssp(v):
    return jnp.log(0.5 + 0.5 * jnp.exp(v))


def _wnet(d, n, s, th_ref):
    th = lambda i: th_ref[i]
    hid = []
    for h in range(HI):
        a = d[0] * th(_O_W1 + ((n * 3 + s) * DF + 0) * HI + h)
        for f in range(1, DF):
            a = a + d[f] * th(_O_W1 + ((n * 3 + s) * DF + f) * HI + h)
        hid.append(_ssp(a + th(_O_B1 + (n * 3 + s) * HI + h)))
    ws = []
    for k in range(KD):
        a = hid[0] * th(_O_W2 + ((n * 3 + s) * HI + 0) * KD + k)
        for h in range(1, HI):
            a = a + hid[h] * th(_O_W2 + ((n * 3 + s) * HI + h) * KD + k)
        ws.append(_ssp(a + th(_O_B2 + (n * 3 + s) * KD + k)))
    return ws


def _body_nuc(dn_ref, th_ref, y2_ref, out_ref):
    d = [dn_ref[f] for f in range(DF)]
    ll = jax.lax.broadcasted_iota(jnp.int32, (128, KD), 0)
    cc = jax.lax.broadcasted_iota(jnp.int32, (128, KD), 1)
    s16 = (ll // NA == cc).astype(jnp.float32)
    for n in range(NI):
        w = _wnet(d, n, 2, th_ref)
        for k in range(KD):
            partial = w[k] * y2_ref[k][None, :]
            out_ref[n, k] = jnp.dot(partial, s16,
                                    preferred_element_type=jnp.float32)


def _body_main(de_ref, zn_ref, th_ref, out_ref):
    th = lambda i: th_ref[i]
    de = [de_ref[f] for f in range(DF)]

    ii = jax.lax.broadcasted_iota(jnp.int32, (NE, 2 * NE), 0)
    ll = jax.lax.broadcasted_iota(jnp.int32, (NE, 2 * NE), 1)
    jm = ll % NE
    sameb = (ii < N_UP) == (jm < N_UP)
    diagm = ii == jm
    l1 = jax.lax.broadcasted_iota(jnp.int32, (2 * NE, 2), 0)
    c1 = jax.lax.broadcasted_iota(jnp.int32, (2 * NE, 2), 1)
    s2 = (l1 // NE == c1).astype(jnp.float32)

    x = [jnp.full((BBH, 2 * NE), th(_O_X0 + e), jnp.float32)
         for e in range(ED)]

    for n in range(NI):
        w_same = _wnet(de, n, 0, th_ref)
        w_anti = _wnet(de, n, 1, th_ref)

        hk = []
        for k in range(KD):
            a = x[0] * th(_O_HW + (n * ED + 0) * KD + k)
            for e in range(1, ED):
                a = a + x[e] * th(_O_HW + (n * ED + e) * KD + k)
            hk.append(a + th(_O_HB + n * KD + k))

        zk = []
        for k in range(KD):
            wc = jnp.where(diagm[None], 0.0,
                           jnp.where(sameb[None], w_same[k], w_anti[k]))
            partial = wc * hk[k][:, None, :]
            z0 = jnp.sum(partial[:, :, :NE], axis=-1)
            z1 = jnp.sum(partial[:, :, NE:], axis=-1)
            zk.append(jnp.concatenate([z0, z1], axis=-1) + zn_ref[n, k])

        xn = []
        for e in range(ED):
            a = zk[0] * th(_O_GW + (n * KD + 0) * ED + e)
            for k in range(1, KD):
                a = a + zk[k] * th(_O_GW + (n * KD + k) * ED + e)
            xn.append(x[e] + a + th(_O_GB + n * ED + e))
        x = xn

    tot = jnp.dot(x[0], s2, preferred_element_type=jnp.float32) * th(_O_OW + 0)
    for e in range(1, ED):
        tot = tot + jnp.dot(x[e], s2,
                            preferred_element_type=jnp.float32) * th(_O_OW + e)
    out_ref[...] = tot[None]


def kernel(dists_nuc, dists_elec, W1, b1, W2, b2, hW, hb, gW, gb, X0, Y, oW):
    Bd = dists_elec.shape[0]
    theta = jnp.concatenate([
        W1.ravel(), b1.ravel(), W2.ravel(), b2.ravel(),
        hW.ravel(), hb.ravel(), gW.ravel(), gb.ravel(),
        X0.ravel(), oW.ravel(),
    ]).astype(jnp.float32)

    X = Bd * NE * NA // 128
    rb = min(RB, X)
    dn_f = jnp.transpose(dists_nuc, (3, 0, 1, 2)).reshape(DF, X, 128)
    y2 = jnp.tile(Y.T, (1, 128 // NA))
    znuc = pl.pallas_call(
        _body_nuc,
        grid=(X // rb,),
        in_specs=[
            pl.BlockSpec((DF, rb, 128), lambda i: (0, i, 0)),
            pl.BlockSpec(memory_space=pltpu.SMEM),
            pl.BlockSpec((KD, 128), lambda i: (0, 0)),
        ],
        out_specs=pl.BlockSpec((NI, KD, rb, KD), lambda i: (0, 0, i, 0)),
        out_shape=jax.ShapeDtypeStruct((NI, KD, X, KD), jnp.float32),
    )(dn_f, theta, y2)
    zn_m = znuc.reshape(NI, KD, Bd // 2, 2 * NE)

    de_m = jnp.transpose(
        dists_elec.reshape(Bd // 2, 2, NE, NE, DF), (4, 0, 2, 1, 3)
    ).reshape(DF, Bd // 2, NE, 2 * NE)

    G = Bd // 2 // BBH
    out = pl.pallas_call(
        _body_main,
        grid=(G,),
        in_specs=[
            pl.BlockSpec((DF, BBH, NE, 2 * NE), lambda i: (0, i, 0, 0)),
            pl.BlockSpec((NI, KD, BBH, 2 * NE), lambda i: (0, 0, i, 0)),
            pl.BlockSpec(memory_space=pltpu.SMEM),
        ],
        out_specs=pl.BlockSpec((1, BBH, 2), lambda i: (i, 0, 0)),
        out_shape=jax.ShapeDtypeStruct((G, BBH, 2), jnp.float32),
    )(de_m, zn_m, theta)
    return out.reshape(Bd)
